# Initial kernel scaffold; baseline (speedup 1.0000x reference)
import functools

import jax
import jax.numpy as jnp
from jax import lax
from jax.experimental import pallas as pl
from jax.experimental.pallas import tpu as pltpu
from jax.experimental.pallas import tpu_sc as plsc

N, NC, E, EC, EB, D = 10000, 2000, 320000, 32000, 10000, 128
NPAD = 10240
NCPAD = 2048
EBPAD = 10240
K1 = 80
K3 = 40
KB = 80

_mesh = plsc.VectorSubcoreMesh(core_axis_name="c", subcore_axis_name="s")


def _fill(ref, rows, cols, val):
    def body(i, _):
        for u in range(cols // 16):
            ref[i, pl.ds(u * 16, 16)] = jnp.full((16,), val, jnp.float32)
        return 0
    lax.fori_loop(0, rows, body, 0)


def _sc_body(src_a, dst_a, attr_a, src_c, dst_c, attr_c,
             src_a2c, dst_a2c, src_c2a, dst_c2a, x_hbm, xcl_hbm,
             out_agg_a, out_agg_c2a, out_cnt_c2a,
             out_agg_c, out_agg_a2c, out_cnt_a2c,
             acc, cnt_acc, zbuf, zbuf16, ones,
             idx_s, idx_d, rows_b, attr_b,
             idx_s3, idx_d3, rows3_b, attr3_b, sem_g, sem_a):
    c = lax.axis_index("c")
    s = lax.axis_index("s")
    wid = c * 16 + s

    _fill(zbuf, 160, 128, 0.0)
    _fill(zbuf16, 640, 16, 0.0)
    _fill(ones, KB, 16, 1.0)

    def zero_acc(nrows_total):
        per = nrows_total // 16
        for j in range(per // 160):
            pltpu.sync_copy(zbuf, acc.at[pl.ds(s * per + j * 160, 160)])
        rem = per % 160
        if rem:
            pltpu.sync_copy(zbuf.at[pl.ds(0, rem)],
                            acc.at[pl.ds(s * per + (per // 160) * 160, rem)])

    def zero_cnt(nrows_total):
        per = nrows_total // 16
        pltpu.sync_copy(zbuf16.at[pl.ds(0, per)], cnt_acc.at[pl.ds(s * per, per)])

    def relu_add(rb, ab, k):
        def body(i, _):
            for u in range(D // 16):
                sl = pl.ds(u * 16, 16)
                rb[i, sl] = jnp.maximum(rb[i, sl] + ab[i, sl], 0.0)
            return 0
        lax.fori_loop(0, k, body, 0)

    def gine_phase(src_h, dst_h, attr_h, xtab, ept, k, rb, ab, isx, idx):
        nrounds = ept // k

        def rbody(r, _):
            base = pl.multiple_of(wid * ept + r * k, 8)
            pltpu.sync_copy(src_h.at[pl.ds(base, k)], isx)
            cp_g = pltpu.async_copy(xtab.at[isx], rb, sem_g)
            cp_a = pltpu.async_copy(attr_h.at[pl.ds(base, k)], ab, sem_a)
            pltpu.sync_copy(dst_h.at[pl.ds(base, k)], idx)
            cp_g.wait()
            cp_a.wait()
            relu_add(rb, ab, k)
            pltpu.sync_copy(rb, acc.at[idx], add=True)
            return 0
        lax.fori_loop(0, nrounds, rbody, 0)

    def sage_phase(src_h, dst_h, xtab):
        def rbody(r, _):
            base = pl.multiple_of(wid * (EBPAD // 32) + r * KB, 8)
            pltpu.sync_copy(src_h.at[pl.ds(base, KB)], idx_s)
            cp_g = pltpu.async_copy(xtab.at[idx_s], rows_b, sem_g)
            pltpu.sync_copy(dst_h.at[pl.ds(base, KB)], idx_d)
            cp_g.wait()
            pltpu.sync_copy(rows_b, acc.at[idx_d], add=True)
            pltpu.sync_copy(ones, cnt_acc.at[idx_d], add=True)
            return 0
        lax.fori_loop(0, EBPAD // 32 // KB, rbody, 0)

    def copyout(dst, nrows_total):
        per = nrows_total // 16
        pltpu.sync_copy(acc.at[pl.ds(s * per, per)], dst.at[c, pl.ds(s * per, per)])

    def copyout_cnt(dst, nrows_total):
        per = nrows_total // 16
        pltpu.sync_copy(cnt_acc.at[pl.ds(s * per, per)], dst.at[c, pl.ds(s * per, per)])

    zero_acc(NPAD)
    plsc.subcore_barrier()
    gine_phase(src_a, dst_a, attr_a, x_hbm, E // 32, K1, rows_b, attr_b, idx_s, idx_d)
    plsc.subcore_barrier()
    copyout(out_agg_a, NPAD)
    plsc.subcore_barrier()

    zero_acc(NPAD)
    zero_cnt(NPAD)
    plsc.subcore_barrier()
    sage_phase(src_c2a, dst_c2a, xcl_hbm)
    plsc.subcore_barrier()
    copyout(out_agg_c2a, NPAD)
    copyout_cnt(out_cnt_c2a, NPAD)
    plsc.subcore_barrier()

    zero_acc(NCPAD)
    plsc.subcore_barrier()
    gine_phase(src_c, dst_c, attr_c, xcl_hbm, EC // 32, K3, rows3_b, attr3_b,
               idx_s3, idx_d3)
    plsc.subcore_barrier()
    copyout(out_agg_c, NCPAD)
    plsc.subcore_barrier()

    zero_acc(NCPAD)
    zero_cnt(NCPAD)
    plsc.subcore_barrier()
    sage_phase(src_a2c, dst_a2c, x_hbm)
    plsc.subcore_barrier()
    copyout(out_agg_a2c, NCPAD)
    copyout_cnt(out_cnt_a2c, NCPAD)


_sc_aggregate = functools.partial(
    pl.kernel,
    out_type=[
        jax.ShapeDtypeStruct((2, NPAD, D), jnp.float32),
        jax.ShapeDtypeStruct((2, NPAD, D), jnp.float32),
        jax.ShapeDtypeStruct((2, NPAD, 16), jnp.float32),
        jax.ShapeDtypeStruct((2, NCPAD, D), jnp.float32),
        jax.ShapeDtypeStruct((2, NCPAD, D), jnp.float32),
        jax.ShapeDtypeStruct((2, NCPAD, 16), jnp.float32),
    ],
    mesh=_mesh,
    scratch_types=[
        pltpu.VMEM_SHARED((NPAD, D), jnp.float32),
        pltpu.VMEM_SHARED((NPAD, 16), jnp.float32),
        pltpu.VMEM((160, D), jnp.float32),
        pltpu.VMEM((640, 16), jnp.float32),
        pltpu.VMEM((KB, 16), jnp.float32),
        pltpu.VMEM((K1,), jnp.int32),
        pltpu.VMEM((K1,), jnp.int32),
        pltpu.VMEM((K1, D), jnp.float32),
        pltpu.VMEM((K1, D), jnp.float32),
        pltpu.VMEM((K3,), jnp.int32),
        pltpu.VMEM((K3,), jnp.int32),
        pltpu.VMEM((K3, D), jnp.float32),
        pltpu.VMEM((K3, D), jnp.float32),
        pltpu.SemaphoreType.DMA,
        pltpu.SemaphoreType.DMA,
    ],
)(_sc_body)


def _dotT(a, w):
    return lax.dot_general(a, w, (((1,), (1,)), ((), ())),
                           preferred_element_type=jnp.float32)


def _tc_body(x_ref, xcl_ref, agg_a_ref, agg_c2a_ref, cnt_c2a_ref,
             agg_c_ref, agg_a2c_ref, cnt_a2c_ref,
             eps_a_ref, W_a_ref, b_a_ref, g_a_ref, be_a_ref,
             eps_c_ref, W_c_ref, b_c_ref, g_c_ref, be_c_ref,
             Wl_a2c_ref, bl_a2c_ref, Wr_a2c_ref,
             Wl_c2a_ref, bl_c2a_ref, Wr_c2a_ref,
             Wm_a_ref, bm_a_ref, Wm_c_ref, bm_c_ref,
             h_ref, hcl_ref):
    x = x_ref[...]
    xcl = xcl_ref[...]

    def gine_mlp(pre, W, b, g, be):
        z = _dotT(pre, W) + b
        mu = jnp.mean(z, axis=0, keepdims=True)
        d = z - mu
        var = jnp.mean(d * d, axis=0, keepdims=True)
        zn = d * lax.rsqrt(var + 1e-5)
        return jnp.maximum(g * zn + be, 0.0)

    agg_a = agg_a_ref[0, :N, :] + agg_a_ref[1, :N, :]
    h1 = gine_mlp((1.0 + eps_a_ref[0, 0]) * x + agg_a,
                  W_a_ref[...], b_a_ref[...], g_a_ref[...], be_a_ref[...])

    agg_c = agg_c_ref[0, :NC, :] + agg_c_ref[1, :NC, :]
    h1c = gine_mlp((1.0 + eps_c_ref[0, 0]) * xcl + agg_c,
                   W_c_ref[...], b_c_ref[...], g_c_ref[...], be_c_ref[...])

    cnt = cnt_c2a_ref[0, :N, 0:1] + cnt_c2a_ref[1, :N, 0:1]
    mean = (agg_c2a_ref[0, :N, :] + agg_c2a_ref[1, :N, :]) / jnp.maximum(cnt, 1.0)
    h_c2a = _dotT(mean, Wl_c2a_ref[...]) + bl_c2a_ref[...] + _dotT(x, Wr_c2a_ref[...])

    cntc = cnt_a2c_ref[0, :NC, 0:1] + cnt_a2c_ref[1, :NC, 0:1]
    meanc = (agg_a2c_ref[0, :NC, :] + agg_a2c_ref[1, :NC, :]) / jnp.maximum(cntc, 1.0)
    h_a2c = _dotT(meanc, Wl_a2c_ref[...]) + bl_a2c_ref[...] + _dotT(xcl, Wr_a2c_ref[...])

    h_ref[...] = _dotT(h1 + h_c2a, Wm_a_ref[...]) + bm_a_ref[...]
    hcl_ref[...] = _dotT(h1c + h_a2c, Wm_c_ref[...]) + bm_c_ref[...]


_tc_merge = pl.pallas_call(
    _tc_body,
    out_shape=[
        jax.ShapeDtypeStruct((N, D), jnp.float32),
        jax.ShapeDtypeStruct((NC, D), jnp.float32),
    ],
)


@jax.jit
def kernel(x, edge_index, edge_attr, x_cl, c2c_edge_index, c2c_edge_attr,
           atom2c_edge_index, c2atom_edge_index,
           eps_a, W_a, b_a, g_a, be_a,
           eps_c, W_c, b_c, g_c, be_c,
           Wl_a2c, bl_a2c, Wr_a2c,
           Wl_c2a, bl_c2a, Wr_c2a,
           Wm_a, bm_a, Wm_c, bm_c):
    src_a, dst_a = edge_index[0], edge_index[1]
    src_c, dst_c = c2c_edge_index[0], c2c_edge_index[1]
    pad = EBPAD - EB
    src_a2c = jnp.concatenate([atom2c_edge_index[0], jnp.zeros((pad,), jnp.int32)])
    dst_a2c = jnp.concatenate([atom2c_edge_index[1], jnp.full((pad,), NC, jnp.int32)])
    src_c2a = jnp.concatenate([c2atom_edge_index[0], jnp.zeros((pad,), jnp.int32)])
    dst_c2a = jnp.concatenate([c2atom_edge_index[1], jnp.full((pad,), N, jnp.int32)])

    agg_a, agg_c2a, cnt_c2a, agg_c, agg_a2c, cnt_a2c = _sc_aggregate(
        src_a, dst_a, edge_attr, src_c, dst_c, c2c_edge_attr,
        src_a2c, dst_a2c, src_c2a, dst_c2a, x, x_cl)

    h, h_cl = _tc_merge(
        x, x_cl, agg_a, agg_c2a, cnt_c2a, agg_c, agg_a2c, cnt_a2c,
        eps_a.reshape(1, 1), W_a, b_a.reshape(1, D), g_a.reshape(1, D),
        be_a.reshape(1, D),
        eps_c.reshape(1, 1), W_c, b_c.reshape(1, D), g_c.reshape(1, D),
        be_c.reshape(1, D),
        Wl_a2c, bl_a2c.reshape(1, D), Wr_a2c,
        Wl_c2a, bl_c2a.reshape(1, D), Wr_c2a,
        Wm_a, bm_a.reshape(1, D), Wm_c, bm_c.reshape(1, D))
    return (h, h_cl)


# speedup vs baseline: 3.7438x; 3.7438x over previous
import functools

import jax
import jax.numpy as jnp
from jax import lax
from jax.experimental import pallas as pl
from jax.experimental.pallas import tpu as pltpu
from jax.experimental.pallas import tpu_sc as plsc

N, NC, E, EC, EB, D = 10000, 2000, 320000, 32000, 10000, 128
NPAD = 10016
NCPAD = 2048
EBPAD = 10240
K1 = 80
K3 = 40
KB = 80

_mesh = plsc.VectorSubcoreMesh(core_axis_name="c", subcore_axis_name="s")


def _fill(ref, rows, cols, val):
    def body(i, _):
        for u in range(cols // 16):
            ref[i, pl.ds(u * 16, 16)] = jnp.full((16,), val, jnp.float32)
        return 0
    lax.fori_loop(0, rows, body, 0)


def _sc_body(src_a, dst_a, attr_a, src_c, dst_c, attr_c,
             src_a2c, dst_a2c, src_c2a, dst_c2a, x_hbm, xcl_hbm,
             out_agg_a, out_agg_c2a, out_cnt_c2a,
             out_agg_c, out_agg_a2c, out_cnt_a2c,
             acc, zbuf, obuf,
             idx_s, idx_d, idx_d2, rows_b, attr_b,
             idx_s3, idx_d3, sem_g, sem_a):
    c = lax.axis_index("c")
    s = lax.axis_index("s")
    wid = c * 16 + s

    _fill(zbuf, 80, 128, 0.0)
    _fill(obuf, KB, 128, 1.0)

    def zero_acc(nrows_total):
        per = nrows_total // 16
        for j in range(per // 80):
            pltpu.sync_copy(zbuf, acc.at[pl.ds(s * per + j * 80, 80)])
        rem = per % 80
        if rem:
            pltpu.sync_copy(zbuf.at[pl.ds(0, rem)],
                            acc.at[pl.ds(s * per + (per // 80) * 80, rem)])

    def relu_add(rb, ab, k):
        def body(i, _):
            for u in range(D // 16):
                sl = pl.ds(u * 16, 16)
                rb[i, sl] = jnp.maximum(rb[i, sl] + ab[i, sl], 0.0)
            return 0
        lax.fori_loop(0, k, body, 0)

    def gine_phase(src_h, dst_h, attr_h, xtab, ept, k, rb, ab, isx, idx):
        nrounds = ept // k

        def rbody(r, _):
            base = pl.multiple_of(wid * ept + r * k, 8)
            pltpu.sync_copy(src_h.at[pl.ds(base, k)], isx)
            cp_g = pltpu.async_copy(xtab.at[isx], rb, sem_g)
            cp_a = pltpu.async_copy(attr_h.at[pl.ds(base, k)], ab, sem_a)
            pltpu.sync_copy(dst_h.at[pl.ds(base, k)], idx)
            cp_g.wait()
            cp_a.wait()
            relu_add(rows_b, attr_b, k)
            pltpu.sync_copy(rb, acc.at[idx], add=True)
            return 0
        lax.fori_loop(0, nrounds, rbody, 0)

    def sage_phase(src_h, dst_h, xtab, cnt_off):
        def rbody(r, _):
            base = pl.multiple_of(wid * (EBPAD // 32) + r * KB, 8)
            pltpu.sync_copy(src_h.at[pl.ds(base, KB)], idx_s)
            cp_g = pltpu.async_copy(xtab.at[idx_s], rows_b, sem_g)
            pltpu.sync_copy(dst_h.at[pl.ds(base, KB)], idx_d)
            if cnt_off is not None:
                for j in range(KB // 16):
                    sl = pl.ds(j * 16, 16)
                    idx_d2[sl] = idx_d[sl] + cnt_off
                pltpu.sync_copy(obuf, acc.at[idx_d2], add=True)
            cp_g.wait()
            pltpu.sync_copy(rows_b, acc.at[idx_d], add=True)
            return 0
        lax.fori_loop(0, EBPAD // 32 // KB, rbody, 0)

    def cnt_phase(dst_h):
        def rbody(r, _):
            base = pl.multiple_of(wid * (EBPAD // 32) + r * KB, 8)
            pltpu.sync_copy(dst_h.at[pl.ds(base, KB)], idx_d)
            pltpu.sync_copy(obuf, acc.at[idx_d], add=True)
            return 0
        lax.fori_loop(0, EBPAD // 32 // KB, rbody, 0)

    def copyout(dst, nrows_total, src_off=0):
        per = nrows_total // 16
        pltpu.sync_copy(acc.at[pl.ds(src_off + s * per, per)],
                        dst.at[c, pl.ds(s * per, per)])

    zero_acc(NPAD)
    plsc.subcore_barrier()
    gine_phase(src_a, dst_a, attr_a, x_hbm, E // 32, K1, rows_b, attr_b,
               idx_s, idx_d)
    plsc.subcore_barrier()
    copyout(out_agg_a, NPAD)
    plsc.subcore_barrier()

    zero_acc(NPAD)
    plsc.subcore_barrier()
    sage_phase(src_c2a, dst_c2a, xcl_hbm, None)
    plsc.subcore_barrier()
    copyout(out_agg_c2a, NPAD)
    plsc.subcore_barrier()

    zero_acc(NPAD)
    plsc.subcore_barrier()
    cnt_phase(dst_c2a)
    plsc.subcore_barrier()
    copyout(out_cnt_c2a, NPAD)
    plsc.subcore_barrier()

    zero_acc(NCPAD)
    plsc.subcore_barrier()
    gine_phase(src_c, dst_c, attr_c, xcl_hbm, EC // 32, K3,
               rows_b.at[pl.ds(0, K3)], attr_b.at[pl.ds(0, K3)],
               idx_s3, idx_d3)
    plsc.subcore_barrier()
    copyout(out_agg_c, NCPAD)
    plsc.subcore_barrier()

    zero_acc(2 * NCPAD)
    plsc.subcore_barrier()
    sage_phase(src_a2c, dst_a2c, x_hbm, NCPAD)
    plsc.subcore_barrier()
    copyout(out_agg_a2c, NCPAD)
    copyout(out_cnt_a2c, NCPAD, src_off=NCPAD)


_sc_aggregate = functools.partial(
    pl.kernel,
    out_type=[
        jax.ShapeDtypeStruct((2, NPAD, D), jnp.float32),
        jax.ShapeDtypeStruct((2, NPAD, D), jnp.float32),
        jax.ShapeDtypeStruct((2, NPAD, D), jnp.float32),
        jax.ShapeDtypeStruct((2, NCPAD, D), jnp.float32),
        jax.ShapeDtypeStruct((2, NCPAD, D), jnp.float32),
        jax.ShapeDtypeStruct((2, NCPAD, D), jnp.float32),
    ],
    mesh=_mesh,
    compiler_params=pltpu.CompilerParams(use_tc_tiling_on_sc=False,
                                         needs_layout_passes=False),
    scratch_types=[
        pltpu.VMEM_SHARED((NPAD, D), jnp.float32),
        pltpu.VMEM((80, D), jnp.float32),
        pltpu.VMEM((KB, D), jnp.float32),
        pltpu.VMEM((K1,), jnp.int32),
        pltpu.VMEM((K1,), jnp.int32),
        pltpu.VMEM((KB,), jnp.int32),
        pltpu.VMEM((K1, D), jnp.float32),
        pltpu.VMEM((K1, D), jnp.float32),
        pltpu.VMEM((K3,), jnp.int32),
        pltpu.VMEM((K3,), jnp.int32),
        pltpu.SemaphoreType.DMA,
        pltpu.SemaphoreType.DMA,
    ],
)(_sc_body)


def _dotT(a, w):
    return lax.dot_general(a, w, (((1,), (1,)), ((), ())),
                           preferred_element_type=jnp.float32)


def _tc_body(x_ref, xcl_ref, agg_a_ref, agg_c2a_ref, cnt_c2a_ref,
             agg_c_ref, agg_a2c_ref, cnt_a2c_ref,
             eps_a_ref, W_a_ref, b_a_ref, g_a_ref, be_a_ref,
             eps_c_ref, W_c_ref, b_c_ref, g_c_ref, be_c_ref,
             Wl_a2c_ref, bl_a2c_ref, Wr_a2c_ref,
             Wl_c2a_ref, bl_c2a_ref, Wr_c2a_ref,
             Wm_a_ref, bm_a_ref, Wm_c_ref, bm_c_ref,
             h_ref, hcl_ref):
    x = x_ref[...]
    xcl = xcl_ref[...]

    def gine_mlp(pre, W, b, g, be):
        z = _dotT(pre, W) + b
        mu = jnp.mean(z, axis=0, keepdims=True)
        d = z - mu
        var = jnp.mean(d * d, axis=0, keepdims=True)
        zn = d * lax.rsqrt(var + 1e-5)
        return jnp.maximum(g * zn + be, 0.0)

    agg_a = agg_a_ref[0, :N, :] + agg_a_ref[1, :N, :]
    h1 = gine_mlp((1.0 + eps_a_ref[0, 0]) * x + agg_a,
                  W_a_ref[...], b_a_ref[...], g_a_ref[...], be_a_ref[...])

    agg_c = agg_c_ref[0, :NC, :] + agg_c_ref[1, :NC, :]
    h1c = gine_mlp((1.0 + eps_c_ref[0, 0]) * xcl + agg_c,
                   W_c_ref[...], b_c_ref[...], g_c_ref[...], be_c_ref[...])

    cnt = cnt_c2a_ref[0, :N, 0:1] + cnt_c2a_ref[1, :N, 0:1]
    mean = (agg_c2a_ref[0, :N, :] + agg_c2a_ref[1, :N, :]) / jnp.maximum(cnt, 1.0)
    h_c2a = _dotT(mean, Wl_c2a_ref[...]) + bl_c2a_ref[...] + _dotT(x, Wr_c2a_ref[...])

    cntc = cnt_a2c_ref[0, :NC, 0:1] + cnt_a2c_ref[1, :NC, 0:1]
    meanc = (agg_a2c_ref[0, :NC, :] + agg_a2c_ref[1, :NC, :]) / jnp.maximum(cntc, 1.0)
    h_a2c = _dotT(meanc, Wl_a2c_ref[...]) + bl_a2c_ref[...] + _dotT(xcl, Wr_a2c_ref[...])

    h_ref[...] = _dotT(h1 + h_c2a, Wm_a_ref[...]) + bm_a_ref[...]
    hcl_ref[...] = _dotT(h1c + h_a2c, Wm_c_ref[...]) + bm_c_ref[...]


_tc_merge = pl.pallas_call(
    _tc_body,
    compiler_params=pltpu.CompilerParams(vmem_limit_bytes=100 * 1024 * 1024),
    out_shape=[
        jax.ShapeDtypeStruct((N, D), jnp.float32),
        jax.ShapeDtypeStruct((NC, D), jnp.float32),
    ],
)


@jax.jit
def kernel(x, edge_index, edge_attr, x_cl, c2c_edge_index, c2c_edge_attr,
           atom2c_edge_index, c2atom_edge_index,
           eps_a, W_a, b_a, g_a, be_a,
           eps_c, W_c, b_c, g_c, be_c,
           Wl_a2c, bl_a2c, Wr_a2c,
           Wl_c2a, bl_c2a, Wr_c2a,
           Wm_a, bm_a, Wm_c, bm_c):
    src_a, dst_a = edge_index[0], edge_index[1]
    src_c, dst_c = c2c_edge_index[0], c2c_edge_index[1]
    pad = EBPAD - EB
    src_a2c = jnp.concatenate([atom2c_edge_index[0], jnp.zeros((pad,), jnp.int32)])
    dst_a2c = jnp.concatenate([atom2c_edge_index[1], jnp.full((pad,), NC, jnp.int32)])
    src_c2a = jnp.concatenate([c2atom_edge_index[0], jnp.zeros((pad,), jnp.int32)])
    dst_c2a = jnp.concatenate([c2atom_edge_index[1], jnp.full((pad,), N, jnp.int32)])

    agg_a, agg_c2a, cnt_c2a, agg_c, agg_a2c, cnt_a2c = _sc_aggregate(
        src_a, dst_a, edge_attr, src_c, dst_c, c2c_edge_attr,
        src_a2c, dst_a2c, src_c2a, dst_c2a, x, x_cl)

    h, h_cl = _tc_merge(
        x, x_cl, agg_a, agg_c2a, cnt_c2a, agg_c, agg_a2c, cnt_a2c,
        eps_a.reshape(1, 1), W_a, b_a.reshape(1, D), g_a.reshape(1, D),
        be_a.reshape(1, D),
        eps_c.reshape(1, 1), W_c, b_c.reshape(1, D), g_c.reshape(1, D),
        be_c.reshape(1, D),
        Wl_a2c, bl_a2c.reshape(1, D), Wr_a2c,
        Wl_c2a, bl_c2a.reshape(1, D), Wr_c2a,
        Wm_a, bm_a.reshape(1, D), Wm_c, bm_c.reshape(1, D))
    return (h, h_cl)


# speedup vs baseline: 4.6832x; 1.2509x over previous
import functools

import jax
import jax.numpy as jnp
from jax import lax
from jax.experimental import pallas as pl
from jax.experimental.pallas import tpu as pltpu
from jax.experimental.pallas import tpu_sc as plsc

N, NC, E, EC, EB, D = 10000, 2000, 320000, 32000, 10000, 128
NPAD = 10016
NCPAD = 2048
EBPAD = 10240
K1 = 80
K3 = 40
KB = 40
BR = 5

_mesh = plsc.VectorSubcoreMesh(core_axis_name="c", subcore_axis_name="s")


def _fill(ref, rows, cols, val):
    def body(i, _):
        for u in range(cols // 16):
            ref[i, pl.ds(u * 16, 16)] = jnp.full((16,), val, jnp.float32)
        return 0
    lax.fori_loop(0, rows, body, 0)


def _sc_body(src_a, dst_a, attr_a, src_c, dst_c, attr_c,
             src_a2c, dst_a2c, src_c2a, dst_c2a, x_hbm, xcl_hbm, zeros_hbm,
             out_agg_a, out_agg_c2a, out_cnt_c2a,
             out_agg_c, out_agg_a2c, out_cnt_a2c,
             acc, obuf, rows0, rows1, attr0, attr1,
             isxr0, isxr1, isxr2, isxr3, isxr4,
             idxr0, idxr1, idxr2, idxr3, idxr4,
             idx3d0, idx_sb, idx_d2,
             sem_g0, sem_g1, sem_a0, sem_a1, sem_i):
    c = lax.axis_index("c")
    s = lax.axis_index("s")
    wid = c * 16 + s

    _fill(obuf, KB, 128, 1.0)

    sem_g = [sem_g0, sem_g1]
    sem_a = [sem_a0, sem_a1]
    isxr = [isxr0, isxr1, isxr2, isxr3, isxr4]
    idxr = [idxr0, idxr1, idxr2, idxr3, idxr4]

    def zero_acc(nrows_total):
        per = nrows_total // 16
        pltpu.sync_copy(zeros_hbm.at[pl.ds(0, K1)], rows0)
        for j in range(per // K1):
            pltpu.sync_copy(rows0, acc.at[pl.ds(s * per + j * K1, K1)])
        rem = per % K1
        if rem:
            pltpu.sync_copy(rows0.at[pl.ds(0, rem)],
                            acc.at[pl.ds(s * per + (per // K1) * K1, rem)])

    def relu_add(rb, ab, k):
        def body(i, _):
            for u in range(D // 16):
                sl = pl.ds(u * 16, 16)
                rb[i, sl] = jnp.maximum(rb[i, sl] + ab[i, sl], 0.0)
            return 0
        lax.fori_loop(0, k, body, 0)

    def gine_pipe(src_h, dst_h, attr_h, xtab, ept, k,
                  rows_dma, rows_cmp, attr_dma, attr_cmp, isxr, idxr):
        nrounds = ept // k
        nblocks = nrounds // BR
        kbr = k * BR
        tbase = wid * ept

        def block(b, _):
            base = pl.multiple_of(tbase + b * kbr, 8)
            idx_cps = []
            for i in range(BR):
                off = pl.multiple_of(base + i * k, 8)
                idx_cps.append(pltpu.async_copy(
                    src_h.at[pl.ds(off, k)], isxr[i], sem_i))
                idx_cps.append(pltpu.async_copy(
                    dst_h.at[pl.ds(off, k)], idxr[i], sem_i))
            for cp in idx_cps:
                cp.wait()

            def fire(i):
                sl = i % 2
                g = pltpu.async_copy(xtab.at[isxr[i]], rows_dma[sl],
                                     sem_g[sl])
                a = pltpu.async_copy(attr_h.at[pl.ds(base + i * k, k)],
                                     attr_dma[sl], sem_a[sl])
                return (g, a)

            cps = {0: fire(0), 1: fire(1)}
            for i in range(BR):
                g, a = cps.pop(i)
                g.wait()
                a.wait()
                relu_add(rows_cmp[i % 2], attr_cmp[i % 2], k)
                pltpu.sync_copy(rows_dma[i % 2], acc.at[idxr[i]], add=True)
                if i + 2 < BR:
                    cps[i + 2] = fire(i + 2)
            return 0

        lax.fori_loop(0, nblocks, block, 0)

    def gine_sync(src_h, dst_h, attr_h, xtab, ept, k, rb_dma, rb_cmp,
                  ab_dma, ab_cmp, isxf, idxf):
        def rbody(r, _):
            base = pl.multiple_of(wid * ept + r * k, 8)
            pltpu.sync_copy(src_h.at[pl.ds(base, k)], isxf)
            cp_g = pltpu.async_copy(xtab.at[isxf], rb_dma, sem_g0)
            cp_a = pltpu.async_copy(attr_h.at[pl.ds(base, k)], ab_dma, sem_a0)
            pltpu.sync_copy(dst_h.at[pl.ds(base, k)], idxf)
            cp_g.wait()
            cp_a.wait()
            relu_add(rb_cmp, ab_cmp, k)
            pltpu.sync_copy(rb_dma, acc.at[idxf], add=True)
            return 0
        lax.fori_loop(0, ept // k, rbody, 0)

    def sage_phase(src_h, dst_h, xtab, cnt_off):
        def rbody(r, _):
            base = pl.multiple_of(wid * (EBPAD // 32) + r * KB, 8)
            pltpu.sync_copy(src_h.at[pl.ds(base, KB)], idx_sb)
            cp_g = pltpu.async_copy(xtab.at[idx_sb], rows0.at[pl.ds(0, KB)],
                                    sem_g0)
            pltpu.sync_copy(dst_h.at[pl.ds(base, KB)], idx3d0)
            if cnt_off is not None:
                offs2 = list(range(0, KB - 15, 16))
                if KB % 16:
                    offs2.append(KB - 16)
                for o in offs2:
                    sl = pl.ds(o, 16)
                    idx_d2[sl] = idx3d0[sl] + cnt_off
                pltpu.sync_copy(obuf, acc.at[idx_d2], add=True)
            cp_g.wait()
            pltpu.sync_copy(rows0.at[pl.ds(0, KB)], acc.at[idx3d0], add=True)
            return 0
        lax.fori_loop(0, EBPAD // 32 // KB, rbody, 0)

    def cnt_phase(dst_h):
        def rbody(r, _):
            base = pl.multiple_of(wid * (EBPAD // 32) + r * KB, 8)
            pltpu.sync_copy(dst_h.at[pl.ds(base, KB)], idx3d0)
            pltpu.sync_copy(obuf, acc.at[idx3d0], add=True)
            return 0
        lax.fori_loop(0, EBPAD // 32 // KB, rbody, 0)

    def copyout(dst, nrows_total, src_off=0):
        per = nrows_total // 16
        pltpu.sync_copy(acc.at[pl.ds(src_off + s * per, per)],
                        dst.at[c, pl.ds(s * per, per)])

    zero_acc(NPAD)
    plsc.subcore_barrier()
    gine_pipe(src_a, dst_a, attr_a, x_hbm, E // 32, K1,
              [rows0, rows1], [rows0, rows1], [attr0, attr1], [attr0, attr1],
              isxr, idxr)
    plsc.subcore_barrier()
    copyout(out_agg_a, NPAD)
    plsc.subcore_barrier()

    zero_acc(NPAD)
    plsc.subcore_barrier()
    sage_phase(src_c2a, dst_c2a, xcl_hbm, None)
    plsc.subcore_barrier()
    copyout(out_agg_c2a, NPAD)
    plsc.subcore_barrier()

    zero_acc(NPAD)
    plsc.subcore_barrier()
    cnt_phase(dst_c2a)
    plsc.subcore_barrier()
    copyout(out_cnt_c2a, NPAD)
    plsc.subcore_barrier()

    zero_acc(NCPAD)
    plsc.subcore_barrier()
    gine_sync(src_c, dst_c, attr_c, xcl_hbm, EC // 32, K3,
              rows0.at[pl.ds(0, K3)], rows0,
              attr0.at[pl.ds(0, K3)], attr0,
              idx_sb, idx3d0)
    plsc.subcore_barrier()
    copyout(out_agg_c, NCPAD)
    plsc.subcore_barrier()

    zero_acc(2 * NCPAD)
    plsc.subcore_barrier()
    sage_phase(src_a2c, dst_a2c, x_hbm, NCPAD)
    plsc.subcore_barrier()
    copyout(out_agg_a2c, NCPAD)
    copyout(out_cnt_a2c, NCPAD, src_off=NCPAD)


_sc_aggregate = functools.partial(
    pl.kernel,
    out_type=[
        jax.ShapeDtypeStruct((2, NPAD, D), jnp.float32),
        jax.ShapeDtypeStruct((2, NPAD, D), jnp.float32),
        jax.ShapeDtypeStruct((2, NPAD, D), jnp.float32),
        jax.ShapeDtypeStruct((2, NCPAD, D), jnp.float32),
        jax.ShapeDtypeStruct((2, NCPAD, D), jnp.float32),
        jax.ShapeDtypeStruct((2, NCPAD, D), jnp.float32),
    ],
    mesh=_mesh,
    compiler_params=pltpu.CompilerParams(use_tc_tiling_on_sc=False,
                                         needs_layout_passes=False),
    scratch_types=[
        pltpu.VMEM_SHARED((NPAD, D), jnp.float32),
        pltpu.VMEM((KB, D), jnp.float32),
        pltpu.VMEM((K1, D), jnp.float32),
        pltpu.VMEM((K1, D), jnp.float32),
        pltpu.VMEM((K1, D), jnp.float32),
        pltpu.VMEM((K1, D), jnp.float32),
        pltpu.VMEM((K1,), jnp.int32),
        pltpu.VMEM((K1,), jnp.int32),
        pltpu.VMEM((K1,), jnp.int32),
        pltpu.VMEM((K1,), jnp.int32),
        pltpu.VMEM((K1,), jnp.int32),
        pltpu.VMEM((K1,), jnp.int32),
        pltpu.VMEM((K1,), jnp.int32),
        pltpu.VMEM((K1,), jnp.int32),
        pltpu.VMEM((K1,), jnp.int32),
        pltpu.VMEM((K1,), jnp.int32),
        pltpu.VMEM((KB,), jnp.int32),
        pltpu.VMEM((KB,), jnp.int32),
        pltpu.VMEM((KB,), jnp.int32),
        pltpu.SemaphoreType.DMA,
        pltpu.SemaphoreType.DMA,
        pltpu.SemaphoreType.DMA,
        pltpu.SemaphoreType.DMA,
        pltpu.SemaphoreType.DMA,
    ],
)(_sc_body)


def _dotT(a, w):
    return lax.dot_general(a, w, (((1,), (1,)), ((), ())),
                           preferred_element_type=jnp.float32)


def _tc_body(x_ref, xcl_ref, agg_a_ref, agg_c2a_ref, cnt_c2a_ref,
             agg_c_ref, agg_a2c_ref, cnt_a2c_ref,
             eps_a_ref, W_a_ref, b_a_ref, g_a_ref, be_a_ref,
             eps_c_ref, W_c_ref, b_c_ref, g_c_ref, be_c_ref,
             Wl_a2c_ref, bl_a2c_ref, Wr_a2c_ref,
             Wl_c2a_ref, bl_c2a_ref, Wr_c2a_ref,
             Wm_a_ref, bm_a_ref, Wm_c_ref, bm_c_ref,
             h_ref, hcl_ref):
    x = x_ref[...]
    xcl = xcl_ref[...]

    def gine_mlp(pre, W, b, g, be):
        z = _dotT(pre, W) + b
        mu = jnp.mean(z, axis=0, keepdims=True)
        d = z - mu
        var = jnp.mean(d * d, axis=0, keepdims=True)
        zn = d * lax.rsqrt(var + 1e-5)
        return jnp.maximum(g * zn + be, 0.0)

    agg_a = agg_a_ref[0, :N, :] + agg_a_ref[1, :N, :]
    h1 = gine_mlp((1.0 + eps_a_ref[0, 0]) * x + agg_a,
                  W_a_ref[...], b_a_ref[...], g_a_ref[...], be_a_ref[...])

    agg_c = agg_c_ref[0, :NC, :] + agg_c_ref[1, :NC, :]
    h1c = gine_mlp((1.0 + eps_c_ref[0, 0]) * xcl + agg_c,
                   W_c_ref[...], b_c_ref[...], g_c_ref[...], be_c_ref[...])

    cnt = cnt_c2a_ref[0, :N, 0:1] + cnt_c2a_ref[1, :N, 0:1]
    mean = (agg_c2a_ref[0, :N, :] + agg_c2a_ref[1, :N, :]) / jnp.maximum(cnt, 1.0)
    h_c2a = _dotT(mean, Wl_c2a_ref[...]) + bl_c2a_ref[...] + _dotT(x, Wr_c2a_ref[...])

    cntc = cnt_a2c_ref[0, :NC, 0:1] + cnt_a2c_ref[1, :NC, 0:1]
    meanc = (agg_a2c_ref[0, :NC, :] + agg_a2c_ref[1, :NC, :]) / jnp.maximum(cntc, 1.0)
    h_a2c = _dotT(meanc, Wl_a2c_ref[...]) + bl_a2c_ref[...] + _dotT(xcl, Wr_a2c_ref[...])

    h_ref[...] = _dotT(h1 + h_c2a, Wm_a_ref[...]) + bm_a_ref[...]
    hcl_ref[...] = _dotT(h1c + h_a2c, Wm_c_ref[...]) + bm_c_ref[...]


_tc_merge = pl.pallas_call(
    _tc_body,
    compiler_params=pltpu.CompilerParams(vmem_limit_bytes=100 * 1024 * 1024),
    out_shape=[
        jax.ShapeDtypeStruct((N, D), jnp.float32),
        jax.ShapeDtypeStruct((NC, D), jnp.float32),
    ],
)


@jax.jit
def kernel(x, edge_index, edge_attr, x_cl, c2c_edge_index, c2c_edge_attr,
           atom2c_edge_index, c2atom_edge_index,
           eps_a, W_a, b_a, g_a, be_a,
           eps_c, W_c, b_c, g_c, be_c,
           Wl_a2c, bl_a2c, Wr_a2c,
           Wl_c2a, bl_c2a, Wr_c2a,
           Wm_a, bm_a, Wm_c, bm_c):
    src_a, dst_a = edge_index[0], edge_index[1]
    src_c, dst_c = c2c_edge_index[0], c2c_edge_index[1]
    pad = EBPAD - EB
    src_a2c = jnp.concatenate([atom2c_edge_index[0], jnp.zeros((pad,), jnp.int32)])
    dst_a2c = jnp.concatenate([atom2c_edge_index[1], jnp.full((pad,), NC, jnp.int32)])
    src_c2a = jnp.concatenate([c2atom_edge_index[0], jnp.zeros((pad,), jnp.int32)])
    dst_c2a = jnp.concatenate([c2atom_edge_index[1], jnp.full((pad,), N, jnp.int32)])
    zeros_hbm = jnp.zeros((K1, D), jnp.float32)

    agg_a, agg_c2a, cnt_c2a, agg_c, agg_a2c, cnt_a2c = _sc_aggregate(
        src_a, dst_a, edge_attr, src_c, dst_c, c2c_edge_attr,
        src_a2c, dst_a2c, src_c2a, dst_c2a, x, x_cl, zeros_hbm)

    h, h_cl = _tc_merge(
        x, x_cl, agg_a, agg_c2a, cnt_c2a, agg_c, agg_a2c, cnt_a2c,
        eps_a.reshape(1, 1), W_a, b_a.reshape(1, D), g_a.reshape(1, D),
        be_a.reshape(1, D),
        eps_c.reshape(1, 1), W_c, b_c.reshape(1, D), g_c.reshape(1, D),
        be_c.reshape(1, D),
        Wl_a2c, bl_a2c.reshape(1, D), Wr_a2c,
        Wl_c2a, bl_c2a.reshape(1, D), Wr_c2a,
        Wm_a, bm_a.reshape(1, D), Wm_c, bm_c.reshape(1, D))
    return (h, h_cl)


# speedup vs baseline: 4.9105x; 1.0485x over previous
import functools

import jax
import jax.numpy as jnp
from jax import lax
from jax.experimental import pallas as pl
from jax.experimental.pallas import tpu as pltpu
from jax.experimental.pallas import tpu_sc as plsc

N, NC, E, EC, EB, D = 10000, 2000, 320000, 32000, 10000, 128
NPAD = 10016
NCPAD = 2048
EBPAD = 10240
K1 = 80
K3 = 40
KB = 40
BR = 5

_mesh = plsc.VectorSubcoreMesh(core_axis_name="c", subcore_axis_name="s")


def _fill(ref, rows, cols, val):
    def body(i, _):
        for u in range(cols // 16):
            ref[i, pl.ds(u * 16, 16)] = jnp.full((16,), val, jnp.float32)
        return 0
    lax.fori_loop(0, rows, body, 0)


def _sc_body(src_a, dst_a, attr_a, src_c, dst_c, attr_c,
             src_a2c, dst_a2c, src_c2a, dst_c2a, x_hbm, xcl_hbm, zeros_hbm,
             out_agg_a, out_agg_c2a, out_cnt_c2a,
             out_agg_c, out_agg_a2c, out_cnt_a2c,
             acc, obuf, rows0, rows1, attr0, attr1,
             isxr0, isxr1, isxr2, isxr3, isxr4,
             idxr0, idxr1, idxr2, idxr3, idxr4,
             isc0, isc1, isc2, isc3, isc4,
             idc0, idc1, idc2, idc3, idc4,
             idx3d0, idx_sb, idx_d2,
             sem_g0, sem_g1, sem_a0, sem_a1, sem_i):
    c = lax.axis_index("c")
    s = lax.axis_index("s")
    wid = c * 16 + s

    _fill(obuf, KB, 128, 1.0)

    sem_g = [sem_g0, sem_g1]
    sem_a = [sem_a0, sem_a1]
    isxr = [isxr0, isxr1, isxr2, isxr3, isxr4]
    idxr = [idxr0, idxr1, idxr2, idxr3, idxr4]
    iscr = [isc0, isc1, isc2, isc3, isc4]
    idcr = [idc0, idc1, idc2, idc3, idc4]

    def zero_acc(nrows_total):
        per = nrows_total // 16
        pltpu.sync_copy(zeros_hbm.at[pl.ds(0, K1)], rows0)
        for j in range(per // K1):
            pltpu.sync_copy(rows0, acc.at[pl.ds(s * per + j * K1, K1)])
        rem = per % K1
        if rem:
            pltpu.sync_copy(rows0.at[pl.ds(0, rem)],
                            acc.at[pl.ds(s * per + (per // K1) * K1, rem)])

    def relu_add(rb, ab, k):
        def body(i, _):
            for u in range(D // 16):
                sl = pl.ds(u * 16, 16)
                rb[i, sl] = jnp.maximum(rb[i, sl] + ab[i, sl], 0.0)
            return 0
        lax.fori_loop(0, k, body, 0)

    def gine_pipe(src_h, dst_h, attr_h, xtab, ept, k,
                  rows_dma, rows_cmp, attr_dma, attr_cmp, isxr, idxr):
        nrounds = ept // k
        nblocks = nrounds // BR
        kbr = k * BR
        tbase = wid * ept

        def block(b, _):
            base = pl.multiple_of(tbase + b * kbr, 8)
            idx_cps = []
            for i in range(BR):
                off = pl.multiple_of(base + i * k, 8)
                idx_cps.append(pltpu.async_copy(
                    src_h.at[pl.ds(off, k)], isxr[i], sem_i))
                idx_cps.append(pltpu.async_copy(
                    dst_h.at[pl.ds(off, k)], idxr[i], sem_i))
            for cp in idx_cps:
                cp.wait()

            def fire(i):
                sl = i % 2
                g = pltpu.async_copy(xtab.at[isxr[i]], rows_dma[sl],
                                     sem_g[sl])
                a = pltpu.async_copy(attr_h.at[pl.ds(base + i * k, k)],
                                     attr_dma[sl], sem_a[sl])
                return (g, a)

            cps = {0: fire(0), 1: fire(1)}
            for i in range(BR):
                g, a = cps.pop(i)
                g.wait()
                a.wait()
                relu_add(rows_cmp[i % 2], attr_cmp[i % 2], k)
                pltpu.sync_copy(rows_dma[i % 2], acc.at[idxr[i]], add=True)
                if i + 2 < BR:
                    cps[i + 2] = fire(i + 2)
            return 0

        lax.fori_loop(0, nblocks, block, 0)

    def gine_sync(src_h, dst_h, attr_h, xtab, ept, k, rb_dma, rb_cmp,
                  ab_dma, ab_cmp, isxf, idxf):
        def rbody(r, _):
            base = pl.multiple_of(wid * ept + r * k, 8)
            pltpu.sync_copy(src_h.at[pl.ds(base, k)], isxf)
            cp_g = pltpu.async_copy(xtab.at[isxf], rb_dma, sem_g0)
            cp_a = pltpu.async_copy(attr_h.at[pl.ds(base, k)], ab_dma, sem_a0)
            pltpu.sync_copy(dst_h.at[pl.ds(base, k)], idxf)
            cp_g.wait()
            cp_a.wait()
            relu_add(rb_cmp, ab_cmp, k)
            pltpu.sync_copy(rb_dma, acc.at[idxf], add=True)
            return 0
        lax.fori_loop(0, ept // k, rbody, 0)

    def sage_phase(src_h, dst_h, xtab, cnt_off):
        def rbody(r, _):
            base = pl.multiple_of(wid * (EBPAD // 32) + r * KB, 8)
            pltpu.sync_copy(src_h.at[pl.ds(base, KB)], idx_sb)
            cp_g = pltpu.async_copy(xtab.at[idx_sb], rows0.at[pl.ds(0, KB)],
                                    sem_g0)
            pltpu.sync_copy(dst_h.at[pl.ds(base, KB)], idx3d0)
            if cnt_off is not None:
                offs2 = list(range(0, KB - 15, 16))
                if KB % 16:
                    offs2.append(KB - 16)
                for o in offs2:
                    sl = pl.ds(o, 16)
                    idx_d2[sl] = idx3d0[sl] + cnt_off
                pltpu.sync_copy(obuf, acc.at[idx_d2], add=True)
            cp_g.wait()
            pltpu.sync_copy(rows0.at[pl.ds(0, KB)], acc.at[idx3d0], add=True)
            return 0
        lax.fori_loop(0, EBPAD // 32 // KB, rbody, 0)

    def cnt_phase(dst_h):
        def rbody(r, _):
            base = pl.multiple_of(wid * (EBPAD // 32) + r * KB, 8)
            pltpu.sync_copy(dst_h.at[pl.ds(base, KB)], idx3d0)
            pltpu.sync_copy(obuf, acc.at[idx3d0], add=True)
            return 0
        lax.fori_loop(0, EBPAD // 32 // KB, rbody, 0)

    def copyout(dst, nrows_total, src_off=0):
        per = nrows_total // 16
        pltpu.sync_copy(acc.at[pl.ds(src_off + s * per, per)],
                        dst.at[c, pl.ds(s * per, per)])

    zero_acc(NPAD)
    plsc.subcore_barrier()
    gine_pipe(src_a, dst_a, attr_a, x_hbm, E // 32, K1,
              [rows0, rows1], [rows0, rows1], [attr0, attr1], [attr0, attr1],
              isxr, idxr)
    plsc.subcore_barrier()
    copyout(out_agg_a, NPAD)
    plsc.subcore_barrier()

    zero_acc(NPAD)
    plsc.subcore_barrier()
    sage_phase(src_c2a, dst_c2a, xcl_hbm, None)
    plsc.subcore_barrier()
    copyout(out_agg_c2a, NPAD)
    plsc.subcore_barrier()

    zero_acc(NPAD)
    plsc.subcore_barrier()
    cnt_phase(dst_c2a)
    plsc.subcore_barrier()
    copyout(out_cnt_c2a, NPAD)
    plsc.subcore_barrier()

    zero_acc(NCPAD)
    plsc.subcore_barrier()
    gine_pipe(src_c, dst_c, attr_c, xcl_hbm, EC // 32, K3,
              [rows0.at[pl.ds(0, K3)], rows1.at[pl.ds(0, K3)]],
              [rows0, rows1],
              [attr0.at[pl.ds(0, K3)], attr1.at[pl.ds(0, K3)]],
              [attr0, attr1],
              iscr, idcr)
    plsc.subcore_barrier()
    copyout(out_agg_c, NCPAD)
    plsc.subcore_barrier()

    zero_acc(2 * NCPAD)
    plsc.subcore_barrier()
    sage_phase(src_a2c, dst_a2c, x_hbm, NCPAD)
    plsc.subcore_barrier()
    copyout(out_agg_a2c, NCPAD)
    copyout(out_cnt_a2c, NCPAD, src_off=NCPAD)


_sc_aggregate = functools.partial(
    pl.kernel,
    out_type=[
        jax.ShapeDtypeStruct((2, NPAD, D), jnp.float32),
        jax.ShapeDtypeStruct((2, NPAD, D), jnp.float32),
        jax.ShapeDtypeStruct((2, NPAD, D), jnp.float32),
        jax.ShapeDtypeStruct((2, NCPAD, D), jnp.float32),
        jax.ShapeDtypeStruct((2, NCPAD, D), jnp.float32),
        jax.ShapeDtypeStruct((2, NCPAD, D), jnp.float32),
    ],
    mesh=_mesh,
    compiler_params=pltpu.CompilerParams(use_tc_tiling_on_sc=False,
                                         needs_layout_passes=False),
    scratch_types=[
        pltpu.VMEM_SHARED((NPAD, D), jnp.float32),
        pltpu.VMEM((KB, D), jnp.float32),
        pltpu.VMEM((K1, D), jnp.float32),
        pltpu.VMEM((K1, D), jnp.float32),
        pltpu.VMEM((K1, D), jnp.float32),
        pltpu.VMEM((K1, D), jnp.float32),
        pltpu.VMEM((K1,), jnp.int32),
        pltpu.VMEM((K1,), jnp.int32),
        pltpu.VMEM((K1,), jnp.int32),
        pltpu.VMEM((K1,), jnp.int32),
        pltpu.VMEM((K1,), jnp.int32),
        pltpu.VMEM((K1,), jnp.int32),
        pltpu.VMEM((K1,), jnp.int32),
        pltpu.VMEM((K1,), jnp.int32),
        pltpu.VMEM((K1,), jnp.int32),
        pltpu.VMEM((K1,), jnp.int32),
        pltpu.VMEM((K3,), jnp.int32),
        pltpu.VMEM((K3,), jnp.int32),
        pltpu.VMEM((K3,), jnp.int32),
        pltpu.VMEM((K3,), jnp.int32),
        pltpu.VMEM((K3,), jnp.int32),
        pltpu.VMEM((K3,), jnp.int32),
        pltpu.VMEM((K3,), jnp.int32),
        pltpu.VMEM((K3,), jnp.int32),
        pltpu.VMEM((K3,), jnp.int32),
        pltpu.VMEM((K3,), jnp.int32),
        pltpu.VMEM((KB,), jnp.int32),
        pltpu.VMEM((KB,), jnp.int32),
        pltpu.VMEM((KB,), jnp.int32),
        pltpu.SemaphoreType.DMA,
        pltpu.SemaphoreType.DMA,
        pltpu.SemaphoreType.DMA,
        pltpu.SemaphoreType.DMA,
        pltpu.SemaphoreType.DMA,
    ],
)(_sc_body)


def _dotT(a, w):
    return lax.dot_general(a, w, (((1,), (1,)), ((), ())),
                           preferred_element_type=jnp.float32)


def _tc_body(x_ref, xcl_ref, agg_a_ref, agg_c2a_ref, cnt_c2a_ref,
             agg_c_ref, agg_a2c_ref, cnt_a2c_ref,
             eps_a_ref, W_a_ref, b_a_ref, g_a_ref, be_a_ref,
             eps_c_ref, W_c_ref, b_c_ref, g_c_ref, be_c_ref,
             Wl_a2c_ref, bl_a2c_ref, Wr_a2c_ref,
             Wl_c2a_ref, bl_c2a_ref, Wr_c2a_ref,
             Wm_a_ref, bm_a_ref, Wm_c_ref, bm_c_ref,
             h_ref, hcl_ref):
    x = x_ref[...]
    xcl = xcl_ref[...]

    def gine_mlp(pre, W, b, g, be):
        z = _dotT(pre, W) + b
        mu = jnp.mean(z, axis=0, keepdims=True)
        d = z - mu
        var = jnp.mean(d * d, axis=0, keepdims=True)
        zn = d * lax.rsqrt(var + 1e-5)
        return jnp.maximum(g * zn + be, 0.0)

    agg_a = agg_a_ref[0, :N, :] + agg_a_ref[1, :N, :]
    h1 = gine_mlp((1.0 + eps_a_ref[0, 0]) * x + agg_a,
                  W_a_ref[...], b_a_ref[...], g_a_ref[...], be_a_ref[...])

    agg_c = agg_c_ref[0, :NC, :] + agg_c_ref[1, :NC, :]
    h1c = gine_mlp((1.0 + eps_c_ref[0, 0]) * xcl + agg_c,
                   W_c_ref[...], b_c_ref[...], g_c_ref[...], be_c_ref[...])

    cnt = cnt_c2a_ref[0, :N, 0:1] + cnt_c2a_ref[1, :N, 0:1]
    mean = (agg_c2a_ref[0, :N, :] + agg_c2a_ref[1, :N, :]) / jnp.maximum(cnt, 1.0)
    h_c2a = _dotT(mean, Wl_c2a_ref[...]) + bl_c2a_ref[...] + _dotT(x, Wr_c2a_ref[...])

    cntc = cnt_a2c_ref[0, :NC, 0:1] + cnt_a2c_ref[1, :NC, 0:1]
    meanc = (agg_a2c_ref[0, :NC, :] + agg_a2c_ref[1, :NC, :]) / jnp.maximum(cntc, 1.0)
    h_a2c = _dotT(meanc, Wl_a2c_ref[...]) + bl_a2c_ref[...] + _dotT(xcl, Wr_a2c_ref[...])

    h_ref[...] = _dotT(h1 + h_c2a, Wm_a_ref[...]) + bm_a_ref[...]
    hcl_ref[...] = _dotT(h1c + h_a2c, Wm_c_ref[...]) + bm_c_ref[...]


_tc_merge = pl.pallas_call(
    _tc_body,
    compiler_params=pltpu.CompilerParams(vmem_limit_bytes=100 * 1024 * 1024),
    out_shape=[
        jax.ShapeDtypeStruct((N, D), jnp.float32),
        jax.ShapeDtypeStruct((NC, D), jnp.float32),
    ],
)


@jax.jit
def kernel(x, edge_index, edge_attr, x_cl, c2c_edge_index, c2c_edge_attr,
           atom2c_edge_index, c2atom_edge_index,
           eps_a, W_a, b_a, g_a, be_a,
           eps_c, W_c, b_c, g_c, be_c,
           Wl_a2c, bl_a2c, Wr_a2c,
           Wl_c2a, bl_c2a, Wr_c2a,
           Wm_a, bm_a, Wm_c, bm_c):
    src_a, dst_a = edge_index[0], edge_index[1]
    src_c, dst_c = c2c_edge_index[0], c2c_edge_index[1]
    pad = EBPAD - EB
    src_a2c = jnp.concatenate([atom2c_edge_index[0], jnp.zeros((pad,), jnp.int32)])
    dst_a2c = jnp.concatenate([atom2c_edge_index[1], jnp.full((pad,), NC, jnp.int32)])
    src_c2a = jnp.concatenate([c2atom_edge_index[0], jnp.zeros((pad,), jnp.int32)])
    dst_c2a = jnp.concatenate([c2atom_edge_index[1], jnp.full((pad,), N, jnp.int32)])
    zeros_hbm = jnp.zeros((K1, D), jnp.float32)

    agg_a, agg_c2a, cnt_c2a, agg_c, agg_a2c, cnt_a2c = _sc_aggregate(
        src_a, dst_a, edge_attr, src_c, dst_c, c2c_edge_attr,
        src_a2c, dst_a2c, src_c2a, dst_c2a, x, x_cl, zeros_hbm)

    h, h_cl = _tc_merge(
        x, x_cl, agg_a, agg_c2a, cnt_c2a, agg_c, agg_a2c, cnt_a2c,
        eps_a.reshape(1, 1), W_a, b_a.reshape(1, D), g_a.reshape(1, D),
        be_a.reshape(1, D),
        eps_c.reshape(1, 1), W_c, b_c.reshape(1, D), g_c.reshape(1, D),
        be_c.reshape(1, D),
        Wl_a2c, bl_a2c.reshape(1, D), Wr_a2c,
        Wl_c2a, bl_c2a.reshape(1, D), Wr_c2a,
        Wm_a, bm_a.reshape(1, D), Wm_c, bm_c.reshape(1, D))
    return (h, h_cl)
